# Initial kernel scaffold; baseline (speedup 1.0000x reference)
import jax
import jax.numpy as jnp
from jax.experimental import pallas as pl


def kernel(x):
    raise NotImplementedError("write your pallas kernel here")


# speedup vs baseline: 1.1532x; 1.1532x over previous
import functools

import jax
import jax.numpy as jnp
from jax import lax
from jax.experimental import pallas as pl
from jax.experimental.pallas import tpu as pltpu

_CLUSTER_NUM = 49
_ITER_LIMIT = 15
_CHUNK = 496
HI = lax.Precision.HIGHEST


def _assign_and_slots(dist, med_col, g, K, N):
    f32 = jnp.float32
    iota_kn0 = lax.broadcasted_iota(jnp.int32, (K, N), 0)
    iota_kn1 = lax.broadcasted_iota(jnp.int32, (K, N), 1)

    P = (med_col == iota_kn1).astype(f32)
    d2m = lax.dot_general(P, dist, (((1,), (0,)), ((), ())),
                          preferred_element_type=f32, precision=HI)
    minv = jnp.min(d2m, axis=0, keepdims=True)
    cand = jnp.where(d2m == minv, iota_kn0, K)
    assign = jnp.min(cand, axis=0, keepdims=True)

    member = assign == iota_kn0
    memf = member.astype(f32)
    sizes = jnp.sum(memf, axis=1, keepdims=True)
    lowKK = (lax.broadcasted_iota(jnp.int32, (K, K), 1)
             < lax.broadcasted_iota(jnp.int32, (K, K), 0)).astype(f32)
    offs = lax.dot_general(lowKK, sizes, (((1,), (0,)), ((), ())),
                           preferred_element_type=f32, precision=HI)
    upNN = (lax.broadcasted_iota(jnp.int32, (N, N), 0)
            < lax.broadcasted_iota(jnp.int32, (N, N), 1)).astype(f32)
    rank_mat = lax.dot_general(memf, upNN, (((1,), (0,)), ((), ())),
                               preferred_element_type=f32, precision=HI)
    rank_row = jnp.sum(memf * rank_mat, axis=0, keepdims=True)
    offs_row = jnp.sum(memf * offs, axis=0, keepdims=True)
    gN = (g * N).astype(f32) if hasattr(g, "astype") else jnp.float32(g * N)
    p = gN + offs_row + rank_row
    t = jnp.floor(p / float(_CHUNK))
    plo = gN + offs
    tlo = jnp.floor(plo / float(_CHUNK))
    tlo_row = jnp.sum(memf * tlo, axis=0, keepdims=True)
    slot = (t - tlo_row).astype(jnp.int32)
    kk = assign + K * slot
    return assign, kk, member


def _med_update(S, member, med_prev_col, K, N):
    f32 = jnp.float32
    iota_kn1 = lax.broadcasted_iota(jnp.int32, (K, N), 1)
    costs = jnp.where(member, S, 1e30)
    minc = jnp.min(costs, axis=1, keepdims=True)
    candc = jnp.where(costs == minc, iota_kn1, N)
    new_med = jnp.min(candc, axis=1, keepdims=True)
    has = jnp.max(member.astype(jnp.int32), axis=1, keepdims=True) > 0
    return jnp.where(has, new_med, med_prev_col)


def _row2col(v_row, K):
    f32 = jnp.float32
    ident = (lax.broadcasted_iota(jnp.int32, (K, K), 0)
             == lax.broadcasted_iota(jnp.int32, (K, K), 1)).astype(f32)
    return lax.dot_general(ident, v_row.astype(f32), (((1,), (1,)), ((), ())),
                           preferred_element_type=f32, precision=HI)


def _tc0_body(x_ref, sq_ref, med0_ref, dist_ref, assign_ref, kk_ref,
              *, K, N, W):
    f32 = jnp.float32
    xg = x_ref[0]
    sq_row = sq_ref[0]
    identN = (lax.broadcasted_iota(jnp.int32, (N, N), 0)
              == lax.broadcasted_iota(jnp.int32, (N, N), 1)).astype(f32)
    sq_col = lax.dot_general(identN, sq_row, (((1,), (1,)), ((), ())),
                             preferred_element_type=f32, precision=HI)
    ones_col = jnp.ones((N, 1), f32)
    sq_row_mat = lax.dot_general(ones_col, sq_col, (((1,), (1,)), ((), ())),
                                 preferred_element_type=f32, precision=HI)
    gmat = lax.dot_general(xg, xg, (((1,), (1,)), ((), ())),
                           preferred_element_type=f32)
    d2 = sq_col + sq_row_mat - 2.0 * gmat
    dist = jnp.sqrt(jnp.maximum(d2, 0.0))
    dist_ref[0] = dist
    med_col = _row2col(med0_ref[0], K).astype(jnp.int32)
    g = pl.program_id(0).astype(jnp.float32)
    assign, kk, _ = _assign_and_slots(dist, med_col, g, K, N)
    assign_ref[0] = assign
    kk_ref[0] = kk


def _serial_slot_segsum(dist_ref, kk_ref, s3_scr, K, N):
    s3_scr[...] = jnp.zeros((3 * K, N), jnp.float32)

    def jb(j, carry):
        c = kk_ref[0, 0, j]
        s3_scr[pl.ds(c, 1), :] += dist_ref[0, pl.ds(j, 1), :]
        return carry
    lax.fori_loop(0, N, jb, 0)
    S3 = s3_scr[...]
    return (S3[:K, :] + S3[K:2 * K, :]) + S3[2 * K:3 * K, :]


def _tcmid_body(dist_ref, kk_ref, assign_ref, med_ref,
                medo_ref, assigno_ref, kko_ref, s3_scr, *, K, N):
    f32 = jnp.float32
    dist = dist_ref[0]
    iota_kn0 = lax.broadcasted_iota(jnp.int32, (K, N), 0)
    member = assign_ref[0] == iota_kn0
    S = _serial_slot_segsum(dist_ref, kk_ref, s3_scr, K, N)
    med_prev_col = _row2col(med_ref[0], K).astype(jnp.int32)
    med_col = _med_update(S, member, med_prev_col, K, N).astype(jnp.int32)
    g = pl.program_id(0).astype(jnp.float32)
    assign, kk, _ = _assign_and_slots(dist, med_col, g, K, N)
    medo_ref[0] = _colrow_exact(med_col, K)
    assigno_ref[0] = assign
    kko_ref[0] = kk


def _colrow_exact(v_col, K):
    f32 = jnp.float32
    ident = (lax.broadcasted_iota(jnp.int32, (K, K), 0)
             == lax.broadcasted_iota(jnp.int32, (K, K), 1)).astype(f32)
    return lax.dot_general(v_col.astype(f32), ident, (((0,), (0,)), ((), ())),
                           preferred_element_type=f32, precision=HI).astype(jnp.int32)


def _tcfin_body(x_ref, dist_ref, kk_ref, assign_ref, med_ref, out_ref, s3_scr,
                *, K, N, W):
    f32 = jnp.float32
    iota_kn0 = lax.broadcasted_iota(jnp.int32, (K, N), 0)
    iota_kn1 = lax.broadcasted_iota(jnp.int32, (K, N), 1)
    member = assign_ref[0] == iota_kn0
    S = _serial_slot_segsum(dist_ref, kk_ref, s3_scr, K, N)
    med_prev_col = _row2col(med_ref[0], K).astype(jnp.int32)
    med = _med_update(S, member, med_prev_col, K, N).astype(jnp.int32)
    iota_kk0 = lax.broadcasted_iota(jnp.int32, (K, K), 0)
    iota_kk1 = lax.broadcasted_iota(jnp.int32, (K, K), 1)
    med_row = _colrow_exact(med, K)
    lt = (med_row < med) | ((med_row == med) & (iota_kk1 < iota_kk0))
    rank_col = jnp.sum(lt.astype(jnp.int32), axis=1, keepdims=True)
    rank_row = _colrow_exact(rank_col, K)
    scat = jnp.where(rank_row == iota_kk0, med_row, 0)
    med_sorted = jnp.sum(scat, axis=1, keepdims=True)
    Pout = (med_sorted == iota_kn1).astype(f32)
    out_ref[0] = lax.dot_general(Pout, x_ref[0], (((1,), (0,)), ((), ())),
                                 preferred_element_type=f32, precision=HI)


def kernel(x):
    if x.ndim == 3:
        x = x[None]
    if x.shape[2] % 2 == 1:
        x = x[:, :, 1:]
    B, F, T, W = x.shape
    x = x[:, :80]
    F = x.shape[1]
    num_chunks = F // 10
    chunks = jnp.split(x, num_chunks, axis=1)
    res = jnp.concatenate(chunks, axis=0).reshape(B * 10, num_chunks * T, W)
    G, N, _ = res.shape
    K = num_chunks * _CLUSTER_NUM

    sq = jnp.sum(res * res, axis=-1)
    med0 = jnp.linspace(0, N - 1, K).astype(jnp.int32)
    med0 = jnp.broadcast_to(med0[None, None, :], (1, 1, K))

    dist, assign, kk = pl.pallas_call(
        functools.partial(_tc0_body, K=K, N=N, W=W),
        grid=(G,),
        in_specs=[pl.BlockSpec((1, N, W), lambda g: (g, 0, 0)),
                  pl.BlockSpec((1, 1, N), lambda g: (g, 0, 0)),
                  pl.BlockSpec((1, 1, K), lambda g: (0, 0, 0))],
        out_specs=[pl.BlockSpec((1, N, N), lambda g: (g, 0, 0)),
                   pl.BlockSpec((1, 1, N), lambda g: (g, 0, 0)),
                   pl.BlockSpec((1, 1, N), lambda g: (g, 0, 0))],
        out_shape=[jax.ShapeDtypeStruct((G, N, N), jnp.float32),
                   jax.ShapeDtypeStruct((G, 1, N), jnp.int32),
                   jax.ShapeDtypeStruct((G, 1, N), jnp.int32)],
    )(res, sq[:, None, :], med0)

    med = jnp.broadcast_to(jnp.linspace(0, N - 1, K).astype(jnp.int32)[None, None, :],
                           (G, 1, K))

    tcmid = pl.pallas_call(
        functools.partial(_tcmid_body, K=K, N=N),
        grid=(G,),
        in_specs=[pl.BlockSpec((1, N, N), lambda g: (g, 0, 0)),
                  pl.BlockSpec((1, 1, N), lambda g: (g, 0, 0),
                               memory_space=pltpu.SMEM),
                  pl.BlockSpec((1, 1, N), lambda g: (g, 0, 0)),
                  pl.BlockSpec((1, 1, K), lambda g: (g, 0, 0))],
        out_specs=[pl.BlockSpec((1, 1, K), lambda g: (g, 0, 0)),
                   pl.BlockSpec((1, 1, N), lambda g: (g, 0, 0)),
                   pl.BlockSpec((1, 1, N), lambda g: (g, 0, 0))],
        out_shape=[jax.ShapeDtypeStruct((G, 1, K), jnp.int32),
                   jax.ShapeDtypeStruct((G, 1, N), jnp.int32),
                   jax.ShapeDtypeStruct((G, 1, N), jnp.int32)],
        scratch_shapes=[pltpu.VMEM((3 * K, N), jnp.float32)],
    )

    for _ in range(_ITER_LIMIT - 1):
        med, assign, kk = tcmid(dist, kk, assign, med)

    out = pl.pallas_call(
        functools.partial(_tcfin_body, K=K, N=N, W=W),
        grid=(G,),
        in_specs=[pl.BlockSpec((1, N, W), lambda g: (g, 0, 0)),
                  pl.BlockSpec((1, N, N), lambda g: (g, 0, 0)),
                  pl.BlockSpec((1, 1, N), lambda g: (g, 0, 0),
                               memory_space=pltpu.SMEM),
                  pl.BlockSpec((1, 1, N), lambda g: (g, 0, 0)),
                  pl.BlockSpec((1, 1, K), lambda g: (g, 0, 0))],
        out_specs=pl.BlockSpec((1, K, W), lambda g: (g, 0, 0)),
        out_shape=jax.ShapeDtypeStruct((G, K, W), jnp.float32),
        scratch_shapes=[pltpu.VMEM((3 * K, N), jnp.float32)],
    )(res, dist, kk, assign, med)

    return out.reshape(B, F, _CLUSTER_NUM, W)


# speedup vs baseline: 1.4080x; 1.2209x over previous
import functools

import jax
import jax.numpy as jnp
from jax import lax
from jax.experimental import pallas as pl
from jax.experimental.pallas import tpu as pltpu

_CLUSTER_NUM = 49
_ITER_LIMIT = 15
_CHUNK = 496
HI = lax.Precision.HIGHEST


def _assign_and_slots(dist, med_col, g, K, N):
    f32 = jnp.float32
    iota_kn0 = lax.broadcasted_iota(jnp.int32, (K, N), 0)
    iota_kn1 = lax.broadcasted_iota(jnp.int32, (K, N), 1)

    P = (med_col == iota_kn1).astype(f32)
    d2m = lax.dot_general(P, dist, (((1,), (0,)), ((), ())),
                          preferred_element_type=f32, precision=HI)
    minv = jnp.min(d2m, axis=0, keepdims=True)
    cand = jnp.where(d2m == minv, iota_kn0, K)
    assign = jnp.min(cand, axis=0, keepdims=True)

    member = assign == iota_kn0
    memf = member.astype(f32)
    sizes = jnp.sum(memf, axis=1, keepdims=True)
    lowKK = (lax.broadcasted_iota(jnp.int32, (K, K), 1)
             < lax.broadcasted_iota(jnp.int32, (K, K), 0)).astype(f32)
    offs = lax.dot_general(lowKK, sizes, (((1,), (0,)), ((), ())),
                           preferred_element_type=f32, precision=HI)
    upNN = (lax.broadcasted_iota(jnp.int32, (N, N), 0)
            < lax.broadcasted_iota(jnp.int32, (N, N), 1)).astype(f32)
    rank_mat = lax.dot_general(memf, upNN, (((1,), (0,)), ((), ())),
                               preferred_element_type=f32, precision=HI)
    rank_row = jnp.sum(memf * rank_mat, axis=0, keepdims=True)
    offs_row = jnp.sum(memf * offs, axis=0, keepdims=True)
    gN = (g * N).astype(f32) if hasattr(g, "astype") else jnp.float32(g * N)
    p = gN + offs_row + rank_row
    t = jnp.floor(p / float(_CHUNK))
    plo = gN + offs
    tlo = jnp.floor(plo / float(_CHUNK))
    tlo_row = jnp.sum(memf * tlo, axis=0, keepdims=True)
    slot = (t - tlo_row).astype(jnp.int32)
    kk = assign + K * slot
    return assign, kk, member


def _med_update(S, member, med_prev_col, K, N):
    f32 = jnp.float32
    iota_kn1 = lax.broadcasted_iota(jnp.int32, (K, N), 1)
    costs = jnp.where(member, S, 1e30)
    minc = jnp.min(costs, axis=1, keepdims=True)
    candc = jnp.where(costs == minc, iota_kn1, N)
    new_med = jnp.min(candc, axis=1, keepdims=True)
    has = jnp.max(member.astype(jnp.int32), axis=1, keepdims=True) > 0
    return jnp.where(has, new_med, med_prev_col)


def _row2col(v_row, K):
    f32 = jnp.float32
    ident = (lax.broadcasted_iota(jnp.int32, (K, K), 0)
             == lax.broadcasted_iota(jnp.int32, (K, K), 1)).astype(f32)
    return lax.dot_general(ident, v_row.astype(f32), (((1,), (1,)), ((), ())),
                           preferred_element_type=f32, precision=HI)


def _tc0_body(x_ref, sq_ref, med0_ref, dist_ref, assign_ref, kk_ref,
              *, K, N, W):
    f32 = jnp.float32
    xg = x_ref[0]
    sq_row = sq_ref[0]
    identN = (lax.broadcasted_iota(jnp.int32, (N, N), 0)
              == lax.broadcasted_iota(jnp.int32, (N, N), 1)).astype(f32)
    sq_col = lax.dot_general(identN, sq_row, (((1,), (1,)), ((), ())),
                             preferred_element_type=f32, precision=HI)
    ones_col = jnp.ones((N, 1), f32)
    sq_row_mat = lax.dot_general(ones_col, sq_col, (((1,), (1,)), ((), ())),
                                 preferred_element_type=f32, precision=HI)
    gmat = lax.dot_general(xg, xg, (((1,), (1,)), ((), ())),
                           preferred_element_type=f32)
    d2 = sq_col + sq_row_mat - 2.0 * gmat
    dist = jnp.sqrt(jnp.maximum(d2, 0.0))
    dist_ref[0] = dist
    med_col = _row2col(med0_ref[0], K).astype(jnp.int32)
    g = pl.program_id(0).astype(jnp.float32)
    assign, kk, _ = _assign_and_slots(dist, med_col, g, K, N)
    assign_ref[0] = assign
    kk_ref[0] = kk


def _serial_slot_segsum(dist_ref, kk_ref, s3_scr, K, N):
    s3_scr[...] = jnp.zeros((3 * K, N), jnp.float32)

    def jb(j, carry):
        c = kk_ref[0, 0, j]
        s3_scr[pl.ds(c, 1), :] += dist_ref[0, pl.ds(j, 1), :]
        return carry
    lax.fori_loop(0, N, jb, 0, unroll=8)
    S3 = s3_scr[...]
    return (S3[:K, :] + S3[K:2 * K, :]) + S3[2 * K:3 * K, :]


def _tcmid_body(dist_ref, kk_ref, assign_ref, med_ref,
                medo_ref, assigno_ref, kko_ref, s3_scr, *, K, N):
    f32 = jnp.float32
    dist = dist_ref[0]
    iota_kn0 = lax.broadcasted_iota(jnp.int32, (K, N), 0)
    member = assign_ref[0] == iota_kn0
    S = _serial_slot_segsum(dist_ref, kk_ref, s3_scr, K, N)
    med_prev_col = _row2col(med_ref[0], K).astype(jnp.int32)
    med_col = _med_update(S, member, med_prev_col, K, N).astype(jnp.int32)
    g = pl.program_id(0).astype(jnp.float32)
    assign, kk, _ = _assign_and_slots(dist, med_col, g, K, N)
    medo_ref[0] = _colrow_exact(med_col, K)
    assigno_ref[0] = assign
    kko_ref[0] = kk


def _colrow_exact(v_col, K):
    f32 = jnp.float32
    ident = (lax.broadcasted_iota(jnp.int32, (K, K), 0)
             == lax.broadcasted_iota(jnp.int32, (K, K), 1)).astype(f32)
    return lax.dot_general(v_col.astype(f32), ident, (((0,), (0,)), ((), ())),
                           preferred_element_type=f32, precision=HI).astype(jnp.int32)


def _tcfin_body(x_ref, dist_ref, kk_ref, assign_ref, med_ref, out_ref, s3_scr,
                *, K, N, W):
    f32 = jnp.float32
    iota_kn0 = lax.broadcasted_iota(jnp.int32, (K, N), 0)
    iota_kn1 = lax.broadcasted_iota(jnp.int32, (K, N), 1)
    member = assign_ref[0] == iota_kn0
    S = _serial_slot_segsum(dist_ref, kk_ref, s3_scr, K, N)
    med_prev_col = _row2col(med_ref[0], K).astype(jnp.int32)
    med = _med_update(S, member, med_prev_col, K, N).astype(jnp.int32)
    iota_kk0 = lax.broadcasted_iota(jnp.int32, (K, K), 0)
    iota_kk1 = lax.broadcasted_iota(jnp.int32, (K, K), 1)
    med_row = _colrow_exact(med, K)
    lt = (med_row < med) | ((med_row == med) & (iota_kk1 < iota_kk0))
    rank_col = jnp.sum(lt.astype(jnp.int32), axis=1, keepdims=True)
    rank_row = _colrow_exact(rank_col, K)
    scat = jnp.where(rank_row == iota_kk0, med_row, 0)
    med_sorted = jnp.sum(scat, axis=1, keepdims=True)
    Pout = (med_sorted == iota_kn1).astype(f32)
    out_ref[0] = lax.dot_general(Pout, x_ref[0], (((1,), (0,)), ((), ())),
                                 preferred_element_type=f32, precision=HI)


def kernel(x):
    if x.ndim == 3:
        x = x[None]
    if x.shape[2] % 2 == 1:
        x = x[:, :, 1:]
    B, F, T, W = x.shape
    x = x[:, :80]
    F = x.shape[1]
    num_chunks = F // 10
    chunks = jnp.split(x, num_chunks, axis=1)
    res = jnp.concatenate(chunks, axis=0).reshape(B * 10, num_chunks * T, W)
    G, N, _ = res.shape
    K = num_chunks * _CLUSTER_NUM

    sq = jnp.sum(res * res, axis=-1)
    med0 = jnp.linspace(0, N - 1, K).astype(jnp.int32)
    med0 = jnp.broadcast_to(med0[None, None, :], (1, 1, K))

    dist, assign, kk = pl.pallas_call(
        functools.partial(_tc0_body, K=K, N=N, W=W),
        grid=(G,),
        in_specs=[pl.BlockSpec((1, N, W), lambda g: (g, 0, 0)),
                  pl.BlockSpec((1, 1, N), lambda g: (g, 0, 0)),
                  pl.BlockSpec((1, 1, K), lambda g: (0, 0, 0))],
        out_specs=[pl.BlockSpec((1, N, N), lambda g: (g, 0, 0)),
                   pl.BlockSpec((1, 1, N), lambda g: (g, 0, 0)),
                   pl.BlockSpec((1, 1, N), lambda g: (g, 0, 0))],
        out_shape=[jax.ShapeDtypeStruct((G, N, N), jnp.float32),
                   jax.ShapeDtypeStruct((G, 1, N), jnp.int32),
                   jax.ShapeDtypeStruct((G, 1, N), jnp.int32)],
    )(res, sq[:, None, :], med0)

    med = jnp.broadcast_to(jnp.linspace(0, N - 1, K).astype(jnp.int32)[None, None, :],
                           (G, 1, K))

    tcmid = pl.pallas_call(
        functools.partial(_tcmid_body, K=K, N=N),
        grid=(G,),
        in_specs=[pl.BlockSpec((1, N, N), lambda g: (g, 0, 0)),
                  pl.BlockSpec((1, 1, N), lambda g: (g, 0, 0),
                               memory_space=pltpu.SMEM),
                  pl.BlockSpec((1, 1, N), lambda g: (g, 0, 0)),
                  pl.BlockSpec((1, 1, K), lambda g: (g, 0, 0))],
        out_specs=[pl.BlockSpec((1, 1, K), lambda g: (g, 0, 0)),
                   pl.BlockSpec((1, 1, N), lambda g: (g, 0, 0)),
                   pl.BlockSpec((1, 1, N), lambda g: (g, 0, 0))],
        out_shape=[jax.ShapeDtypeStruct((G, 1, K), jnp.int32),
                   jax.ShapeDtypeStruct((G, 1, N), jnp.int32),
                   jax.ShapeDtypeStruct((G, 1, N), jnp.int32)],
        scratch_shapes=[pltpu.VMEM((3 * K, N), jnp.float32)],
    )

    for _ in range(_ITER_LIMIT - 1):
        med, assign, kk = tcmid(dist, kk, assign, med)

    out = pl.pallas_call(
        functools.partial(_tcfin_body, K=K, N=N, W=W),
        grid=(G,),
        in_specs=[pl.BlockSpec((1, N, W), lambda g: (g, 0, 0)),
                  pl.BlockSpec((1, N, N), lambda g: (g, 0, 0)),
                  pl.BlockSpec((1, 1, N), lambda g: (g, 0, 0),
                               memory_space=pltpu.SMEM),
                  pl.BlockSpec((1, 1, N), lambda g: (g, 0, 0)),
                  pl.BlockSpec((1, 1, K), lambda g: (g, 0, 0))],
        out_specs=pl.BlockSpec((1, K, W), lambda g: (g, 0, 0)),
        out_shape=jax.ShapeDtypeStruct((G, K, W), jnp.float32),
        scratch_shapes=[pltpu.VMEM((3 * K, N), jnp.float32)],
    )(res, dist, kk, assign, med)

    return out.reshape(B, F, _CLUSTER_NUM, W)


# speedup vs baseline: 1.4274x; 1.0138x over previous
import functools

import jax
import jax.numpy as jnp
from jax import lax
from jax.experimental import pallas as pl
from jax.experimental.pallas import tpu as pltpu

_CLUSTER_NUM = 49
_ITER_LIMIT = 15
_CHUNK = 496
HI = lax.Precision.HIGHEST


def _assign_and_slots(dist, med_col, g, K, N):
    f32 = jnp.float32
    iota_kn0 = lax.broadcasted_iota(jnp.int32, (K, N), 0)
    iota_kn1 = lax.broadcasted_iota(jnp.int32, (K, N), 1)

    P = (med_col == iota_kn1).astype(f32)
    d2m = lax.dot_general(P, dist, (((1,), (0,)), ((), ())),
                          preferred_element_type=f32, precision=HI)
    minv = jnp.min(d2m, axis=0, keepdims=True)
    cand = jnp.where(d2m == minv, iota_kn0, K)
    assign = jnp.min(cand, axis=0, keepdims=True)

    member = assign == iota_kn0
    memf = member.astype(f32)
    sizes = jnp.sum(memf, axis=1, keepdims=True)
    lowKK = (lax.broadcasted_iota(jnp.int32, (K, K), 1)
             < lax.broadcasted_iota(jnp.int32, (K, K), 0)).astype(f32)
    offs = lax.dot_general(lowKK, sizes, (((1,), (0,)), ((), ())),
                           preferred_element_type=f32, precision=HI)
    upNN = (lax.broadcasted_iota(jnp.int32, (N, N), 0)
            < lax.broadcasted_iota(jnp.int32, (N, N), 1)).astype(f32)
    rank_mat = lax.dot_general(memf, upNN, (((1,), (0,)), ((), ())),
                               preferred_element_type=f32, precision=HI)
    rank_row = jnp.sum(memf * rank_mat, axis=0, keepdims=True)
    offs_row = jnp.sum(memf * offs, axis=0, keepdims=True)
    gN = (g * N).astype(f32) if hasattr(g, "astype") else jnp.float32(g * N)
    p = gN + offs_row + rank_row
    t = jnp.floor(p / float(_CHUNK))
    plo = gN + offs
    tlo = jnp.floor(plo / float(_CHUNK))
    tlo_row = jnp.sum(memf * tlo, axis=0, keepdims=True)
    slot = (t - tlo_row).astype(jnp.int32)
    kk = assign + K * slot
    return assign, kk, member


def _med_update(S, member, med_prev_col, K, N):
    f32 = jnp.float32
    iota_kn1 = lax.broadcasted_iota(jnp.int32, (K, N), 1)
    costs = jnp.where(member, S, 1e30)
    minc = jnp.min(costs, axis=1, keepdims=True)
    candc = jnp.where(costs == minc, iota_kn1, N)
    new_med = jnp.min(candc, axis=1, keepdims=True)
    has = jnp.max(member.astype(jnp.int32), axis=1, keepdims=True) > 0
    return jnp.where(has, new_med, med_prev_col)


def _row2col(v_row, K):
    f32 = jnp.float32
    ident = (lax.broadcasted_iota(jnp.int32, (K, K), 0)
             == lax.broadcasted_iota(jnp.int32, (K, K), 1)).astype(f32)
    return lax.dot_general(ident, v_row.astype(f32), (((1,), (1,)), ((), ())),
                           preferred_element_type=f32, precision=HI)


def _tc0_body(x_ref, sq_ref, med0_ref, dist_ref, assign_ref, kk_ref,
              *, K, N, W):
    f32 = jnp.float32
    xg = x_ref[0]
    sq_row = sq_ref[0]
    identN = (lax.broadcasted_iota(jnp.int32, (N, N), 0)
              == lax.broadcasted_iota(jnp.int32, (N, N), 1)).astype(f32)
    sq_col = lax.dot_general(identN, sq_row, (((1,), (1,)), ((), ())),
                             preferred_element_type=f32, precision=HI)
    ones_col = jnp.ones((N, 1), f32)
    sq_row_mat = lax.dot_general(ones_col, sq_col, (((1,), (1,)), ((), ())),
                                 preferred_element_type=f32, precision=HI)
    gmat = lax.dot_general(xg, xg, (((1,), (1,)), ((), ())),
                           preferred_element_type=f32)
    d2 = sq_col + sq_row_mat - 2.0 * gmat
    dist = jnp.sqrt(jnp.maximum(d2, 0.0))
    dist_ref[0] = dist
    med_col = _row2col(med0_ref[0], K).astype(jnp.int32)
    g = pl.program_id(0).astype(jnp.float32)
    assign, kk, _ = _assign_and_slots(dist, med_col, g, K, N)
    assign_ref[0] = assign
    kk_ref[0] = kk


def _serial_slot_segsum(dist_ref, kk_ref, s3_scr, K, N):
    s3_scr[...] = jnp.zeros((3 * K, N), jnp.float32)

    def jb(j, carry):
        c = kk_ref[0, 0, j]
        s3_scr[pl.ds(c, 1), :] += dist_ref[0, pl.ds(j, 1), :]
        return carry
    lax.fori_loop(0, N, jb, 0, unroll=16)
    S3 = s3_scr[...]
    return (S3[:K, :] + S3[K:2 * K, :]) + S3[2 * K:3 * K, :]


def _tcmid_body(dist_ref, kk_ref, assign_ref, med_ref,
                medo_ref, assigno_ref, kko_ref, s3_scr, *, K, N):
    f32 = jnp.float32
    dist = dist_ref[0]
    iota_kn0 = lax.broadcasted_iota(jnp.int32, (K, N), 0)
    member = assign_ref[0] == iota_kn0
    S = _serial_slot_segsum(dist_ref, kk_ref, s3_scr, K, N)
    med_prev_col = _row2col(med_ref[0], K).astype(jnp.int32)
    med_col = _med_update(S, member, med_prev_col, K, N).astype(jnp.int32)
    g = pl.program_id(0).astype(jnp.float32)
    assign, kk, _ = _assign_and_slots(dist, med_col, g, K, N)
    medo_ref[0] = _colrow_exact(med_col, K)
    assigno_ref[0] = assign
    kko_ref[0] = kk


def _colrow_exact(v_col, K):
    f32 = jnp.float32
    ident = (lax.broadcasted_iota(jnp.int32, (K, K), 0)
             == lax.broadcasted_iota(jnp.int32, (K, K), 1)).astype(f32)
    return lax.dot_general(v_col.astype(f32), ident, (((0,), (0,)), ((), ())),
                           preferred_element_type=f32, precision=HI).astype(jnp.int32)


def _tcfin_body(x_ref, dist_ref, kk_ref, assign_ref, med_ref, out_ref, s3_scr,
                *, K, N, W):
    f32 = jnp.float32
    iota_kn0 = lax.broadcasted_iota(jnp.int32, (K, N), 0)
    iota_kn1 = lax.broadcasted_iota(jnp.int32, (K, N), 1)
    member = assign_ref[0] == iota_kn0
    S = _serial_slot_segsum(dist_ref, kk_ref, s3_scr, K, N)
    med_prev_col = _row2col(med_ref[0], K).astype(jnp.int32)
    med = _med_update(S, member, med_prev_col, K, N).astype(jnp.int32)
    iota_kk0 = lax.broadcasted_iota(jnp.int32, (K, K), 0)
    iota_kk1 = lax.broadcasted_iota(jnp.int32, (K, K), 1)
    med_row = _colrow_exact(med, K)
    lt = (med_row < med) | ((med_row == med) & (iota_kk1 < iota_kk0))
    rank_col = jnp.sum(lt.astype(jnp.int32), axis=1, keepdims=True)
    rank_row = _colrow_exact(rank_col, K)
    scat = jnp.where(rank_row == iota_kk0, med_row, 0)
    med_sorted = jnp.sum(scat, axis=1, keepdims=True)
    Pout = (med_sorted == iota_kn1).astype(f32)
    out_ref[0] = lax.dot_general(Pout, x_ref[0], (((1,), (0,)), ((), ())),
                                 preferred_element_type=f32, precision=HI)


def kernel(x):
    if x.ndim == 3:
        x = x[None]
    if x.shape[2] % 2 == 1:
        x = x[:, :, 1:]
    B, F, T, W = x.shape
    x = x[:, :80]
    F = x.shape[1]
    num_chunks = F // 10
    chunks = jnp.split(x, num_chunks, axis=1)
    res = jnp.concatenate(chunks, axis=0).reshape(B * 10, num_chunks * T, W)
    G, N, _ = res.shape
    K = num_chunks * _CLUSTER_NUM

    sq = jnp.sum(res * res, axis=-1)
    med0 = jnp.linspace(0, N - 1, K).astype(jnp.int32)
    med0 = jnp.broadcast_to(med0[None, None, :], (1, 1, K))

    dist, assign, kk = pl.pallas_call(
        functools.partial(_tc0_body, K=K, N=N, W=W),
        grid=(G,),
        in_specs=[pl.BlockSpec((1, N, W), lambda g: (g, 0, 0)),
                  pl.BlockSpec((1, 1, N), lambda g: (g, 0, 0)),
                  pl.BlockSpec((1, 1, K), lambda g: (0, 0, 0))],
        out_specs=[pl.BlockSpec((1, N, N), lambda g: (g, 0, 0)),
                   pl.BlockSpec((1, 1, N), lambda g: (g, 0, 0)),
                   pl.BlockSpec((1, 1, N), lambda g: (g, 0, 0))],
        out_shape=[jax.ShapeDtypeStruct((G, N, N), jnp.float32),
                   jax.ShapeDtypeStruct((G, 1, N), jnp.int32),
                   jax.ShapeDtypeStruct((G, 1, N), jnp.int32)],
    )(res, sq[:, None, :], med0)

    med = jnp.broadcast_to(jnp.linspace(0, N - 1, K).astype(jnp.int32)[None, None, :],
                           (G, 1, K))

    tcmid = pl.pallas_call(
        functools.partial(_tcmid_body, K=K, N=N),
        grid=(G,),
        in_specs=[pl.BlockSpec((1, N, N), lambda g: (g, 0, 0)),
                  pl.BlockSpec((1, 1, N), lambda g: (g, 0, 0),
                               memory_space=pltpu.SMEM),
                  pl.BlockSpec((1, 1, N), lambda g: (g, 0, 0)),
                  pl.BlockSpec((1, 1, K), lambda g: (g, 0, 0))],
        out_specs=[pl.BlockSpec((1, 1, K), lambda g: (g, 0, 0)),
                   pl.BlockSpec((1, 1, N), lambda g: (g, 0, 0)),
                   pl.BlockSpec((1, 1, N), lambda g: (g, 0, 0))],
        out_shape=[jax.ShapeDtypeStruct((G, 1, K), jnp.int32),
                   jax.ShapeDtypeStruct((G, 1, N), jnp.int32),
                   jax.ShapeDtypeStruct((G, 1, N), jnp.int32)],
        scratch_shapes=[pltpu.VMEM((3 * K, N), jnp.float32)],
    )

    for _ in range(_ITER_LIMIT - 1):
        med, assign, kk = tcmid(dist, kk, assign, med)

    out = pl.pallas_call(
        functools.partial(_tcfin_body, K=K, N=N, W=W),
        grid=(G,),
        in_specs=[pl.BlockSpec((1, N, W), lambda g: (g, 0, 0)),
                  pl.BlockSpec((1, N, N), lambda g: (g, 0, 0)),
                  pl.BlockSpec((1, 1, N), lambda g: (g, 0, 0),
                               memory_space=pltpu.SMEM),
                  pl.BlockSpec((1, 1, N), lambda g: (g, 0, 0)),
                  pl.BlockSpec((1, 1, K), lambda g: (g, 0, 0))],
        out_specs=pl.BlockSpec((1, K, W), lambda g: (g, 0, 0)),
        out_shape=jax.ShapeDtypeStruct((G, K, W), jnp.float32),
        scratch_shapes=[pltpu.VMEM((3 * K, N), jnp.float32)],
    )(res, dist, kk, assign, med)

    return out.reshape(B, F, _CLUSTER_NUM, W)
